# baseline (speedup 1.0000x reference)
import functools

import jax
import jax.numpy as jnp
from jax import lax
from jax.experimental import pallas as pl
from jax.experimental.pallas import tpu as pltpu
from jax.experimental.pallas import tpu_sc as plsc

_SUB = 128
_NSUB = 4
_CHUNK = _SUB * _NSUB


def _fill_body(n_out, n_chunks, nw, feat_hbm, idx_hbm, mask_hbm, out_hbm,
               idx_v, mask_v, rows_v, sem):
    d = feat_hbm.shape[1]
    wid = lax.axis_index("s") * 2 + lax.axis_index("c")

    def chunk_body(k, carry):
        c = wid + k * nw

        @pl.when(c < n_chunks)
        def _():
            base = jnp.minimum(c * _CHUNK, n_out - _CHUNK)
            for j in range(_NSUB):
                pltpu.sync_copy(idx_hbm.at[pl.ds(base + j * _SUB, _SUB)],
                                idx_v.at[j])
            pltpu.sync_copy(mask_hbm.at[pl.ds(base, _CHUNK)], mask_v)
            copies = [
                pltpu.async_copy(feat_hbm.at[idx_v.at[j]],
                                 rows_v.at[pl.ds(j * _SUB, _SUB)], sem)
                for j in range(_NSUB)
            ]
            for cp in copies:
                cp.wait()

            def group_body(g, carry2):
                m16 = mask_v[pl.ds(g * 16, 16)]
                for i in range(16):
                    @pl.when(m16[i] == 0)
                    def _(i=i):
                        z = jnp.zeros((16,), jnp.float32)
                        for cc in range(d // 16):
                            rows_v[g * 16 + i, pl.ds(cc * 16, 16)] = z

                return carry2

            lax.fori_loop(0, _CHUNK // 16, group_body, 0)
            pltpu.sync_copy(rows_v, out_hbm.at[pl.ds(base, _CHUNK)])

        return carry

    n_my_chunks = (n_chunks + nw - 1) // nw
    lax.fori_loop(0, n_my_chunks, chunk_body, 0)


def kernel(features, out_to_in_idx, out_mask):
    n_in, d = features.shape
    n_out = out_to_in_idx.shape[0]
    idx = out_to_in_idx.astype(jnp.int32)
    mask = out_mask.astype(jnp.int32)

    info = plsc.get_sparse_core_info()
    nw = info.num_cores * info.num_subcores
    n_chunks = (n_out + _CHUNK - 1) // _CHUNK

    mesh = plsc.VectorSubcoreMesh(core_axis_name="c", subcore_axis_name="s")
    run = pl.kernel(
        functools.partial(_fill_body, n_out, n_chunks, nw),
        out_type=jax.ShapeDtypeStruct((n_out, d), jnp.float32),
        mesh=mesh,
        scratch_types=[
            pltpu.VMEM((_NSUB, _SUB), jnp.int32),
            pltpu.VMEM((_CHUNK,), jnp.int32),
            pltpu.VMEM((_CHUNK, d), jnp.float32),
            pltpu.SemaphoreType.DMA,
        ],
        compiler_params=pltpu.CompilerParams(use_tc_tiling_on_sc=False),
    )
    return run(features, idx, mask)
